# baseline (speedup 1.0000x reference)
import functools

import jax
import jax.numpy as jnp
from jax import lax
from jax.experimental import pallas as pl
from jax.experimental.pallas import tpu as pltpu
from jax.experimental.pallas import tpu_sc as plsc

_T = 2048
_D = 1024
_H = 4096
_E = 8
_K = 2
_N = _T * _K
_TILE = 1024
_SUB = 256
_NSUB = _TILE // _SUB
_NT = _N // _TILE + _E
_NP = _NT * _TILE
_HC = 1024
_NH = _H // _HC
_NW = 32


def _route_body(x_ref, wr_ref, br_ref, dest_ref, w_ref, te_ref, tvr_ref):
    logits = lax.dot_general(x_ref[...], wr_ref[...], (((1,), (1,)), ((), ())),
                             preferred_element_type=jnp.float32) + br_ref[...]
    e_iota = lax.broadcasted_iota(jnp.int32, (_T, _E), 1)
    m0 = jnp.max(logits, axis=1, keepdims=True)
    i0 = jnp.min(jnp.where(logits == m0, e_iota, _E), axis=1, keepdims=True)
    masked = jnp.where(e_iota == i0, -jnp.inf, logits)
    m1 = jnp.max(masked, axis=1, keepdims=True)
    i1 = jnp.min(jnp.where(masked == m1, e_iota, _E), axis=1, keepdims=True)
    w0 = 1.0 / (1.0 + jnp.exp(m1 - m0))

    fid = jnp.concatenate([i0, i1], axis=0)
    lane_e = lax.broadcasted_iota(jnp.int32, (1, _E), 1)
    oh = (fid == lane_e).astype(jnp.int32)
    cs = oh
    sh = 1
    while sh < _N:
        cs = cs + jnp.concatenate(
            [jnp.zeros((sh, _E), jnp.int32), cs[:-sh, :]], axis=0)
        sh *= 2
    counts = cs[_N - 1:_N, :]
    rank = jnp.sum(oh * cs, axis=1, keepdims=True) - 1
    ptiles = lax.shift_right_logical(counts + (_TILE - 1), 10)
    pcs = ptiles
    for sh in (1, 2, 4):
        pcs = pcs + jnp.concatenate(
            [jnp.zeros((1, sh), jnp.int32), pcs[:, :-sh]], axis=1)
    gstart = (pcs - ptiles) * _TILE
    dest_ref[...] = jnp.sum(oh * gstart, axis=1, keepdims=True) + rank
    w_ref[...] = jnp.concatenate([w0, 1.0 - w0], axis=0)

    tstart = lax.broadcasted_iota(jnp.int32, (_NT, 1), 0) * _TILE
    te = jnp.clip(jnp.sum((gstart <= tstart).astype(jnp.int32),
                          axis=1, keepdims=True) - 1, 0, _E - 1)
    ohe = (te == lane_e).astype(jnp.int32)
    gct = jnp.sum(ohe * (gstart + counts), axis=1, keepdims=True)
    te_ref[...] = te
    tvr_ref[...] = jnp.clip(gct - tstart, 0, _TILE)


def _route(x2, Wr, br2):
    return pl.pallas_call(
        _route_body,
        out_shape=[
            jax.ShapeDtypeStruct((_N, 1), jnp.int32),
            jax.ShapeDtypeStruct((_N, 1), jnp.float32),
            jax.ShapeDtypeStruct((_NT, 1), jnp.int32),
            jax.ShapeDtypeStruct((_NT, 1), jnp.int32),
        ],
    )(x2, Wr, br2)


_SLOTS = _N // _NW
_SCH = 32


def _sc_dispatch(x2, dest2):
    mesh = plsc.VectorSubcoreMesh(core_axis_name="c", subcore_axis_name="s")

    @functools.partial(
        pl.kernel,
        mesh=mesh,
        out_type=jax.ShapeDtypeStruct((_NP, _D), jnp.float32),
        scratch_types=[
            pltpu.VMEM((_SCH,), jnp.int32),
            pltpu.VMEM((_SCH,), jnp.int32),
            pltpu.VMEM((_SCH, _D), jnp.float32),
            pltpu.VMEM((_SCH, _D), jnp.float32),
            pltpu.SemaphoreType.DMA,
            pltpu.SemaphoreType.DMA,
        ],
    )
    def k(x_hbm, dest_hbm, xs_hbm, i0_v, i1_v, b0, b1, s0, s1):
        wid = lax.axis_index("s") * 2 + lax.axis_index("c")
        kk = wid // 16
        t0 = (wid % 16) * _SLOTS
        idxs = (i0_v, i1_v)
        bufs = (b0, b1)
        sems = (s0, s1)
        cps = [None, None]
        for c in range(_SLOTS // _SCH):
            b = c % 2
            if cps[b] is not None:
                cps[b].wait()
            pltpu.sync_copy(dest_hbm.at[kk, pl.ds(t0 + c * _SCH, _SCH)],
                            idxs[b])
            pltpu.sync_copy(x_hbm.at[pl.ds(t0 + c * _SCH, _SCH)], bufs[b])
            cps[b] = pltpu.async_copy(bufs[b], xs_hbm.at[idxs[b]], sems[b])
        for cp in cps:
            if cp is not None:
                cp.wait()

    return k(x2, dest2)


def _ffn_body(te_ref, tvr_ref, xs_ref, w1a_ref, w1b_ref, b1a_ref, b1b_ref,
              w2_ref, b2_ref, out_ref):
    i = pl.program_id(0)
    j = pl.program_id(1)
    w1a = w1a_ref[0].astype(jnp.bfloat16)
    w1b = w1b_ref[0].astype(jnp.bfloat16)
    w2 = w2_ref[0].astype(jnp.bfloat16)

    for k in range(_NSUB):
        @pl.when(tvr_ref[i] > k * _SUB)
        def _():
            rows = pl.ds(k * _SUB, _SUB)
            xb = xs_ref[rows, :].astype(jnp.bfloat16)
            h1 = lax.dot_general(xb, w1a, (((1,), (1,)), ((), ())),
                                 preferred_element_type=jnp.float32) + b1a_ref[0]
            h2 = lax.dot_general(xb, w1b, (((1,), (1,)), ((), ())),
                                 preferred_element_type=jnp.float32) + b1b_ref[0]
            a = (h1 * lax.logistic(h1) * h2).astype(jnp.bfloat16)
            part = lax.dot_general(a, w2, (((1,), (1,)), ((), ())),
                                   preferred_element_type=jnp.float32)

            @pl.when(j == 0)
            def _():
                out_ref[rows, :] = part

            @pl.when(j > 0)
            def _():
                out_ref[rows, :] = out_ref[rows, :] + part

            @pl.when(j == _NH - 1)
            def _():
                out_ref[rows, :] = out_ref[rows, :] + b2_ref[0]


def _ffn(xs, W1, b1, W2, b2, te, tvr):
    grid_spec = pltpu.PrefetchScalarGridSpec(
        num_scalar_prefetch=2,
        grid=(_NT, _NH),
        in_specs=[
            pl.BlockSpec((_TILE, _D), lambda i, j, te, tv: (i, 0)),
            pl.BlockSpec((1, _HC, _D), lambda i, j, te, tv: (te[i], j, 0)),
            pl.BlockSpec((1, _HC, _D), lambda i, j, te, tv: (te[i], _NH + j, 0)),
            pl.BlockSpec((1, 1, _HC), lambda i, j, te, tv: (te[i], 0, j)),
            pl.BlockSpec((1, 1, _HC), lambda i, j, te, tv: (te[i], 0, _NH + j)),
            pl.BlockSpec((1, _D, _HC), lambda i, j, te, tv: (te[i], 0, j)),
            pl.BlockSpec((1, 1, _D), lambda i, j, te, tv: (te[i], 0, 0)),
        ],
        out_specs=pl.BlockSpec((_TILE, _D), lambda i, j, te, tv: (i, 0)),
    )
    return pl.pallas_call(
        _ffn_body,
        grid_spec=grid_spec,
        out_shape=jax.ShapeDtypeStruct((_NP, _D), jnp.float32),
    )(te, tvr, xs, W1, W1, b1.reshape(_E, 1, 2 * _H), b1.reshape(_E, 1, 2 * _H),
      W2, b2.reshape(_E, 1, _D))


_CROWS = _T // _NW
_CCH = 32


def _sc_combine(ys, dest2):
    mesh = plsc.VectorSubcoreMesh(core_axis_name="c", subcore_axis_name="s")

    @functools.partial(
        pl.kernel,
        mesh=mesh,
        out_type=[
            jax.ShapeDtypeStruct((_T, _D), jnp.float32),
            jax.ShapeDtypeStruct((_T, _D), jnp.float32),
        ],
        scratch_types=[
            pltpu.VMEM((_CCH,), jnp.int32),
            pltpu.VMEM((_CCH,), jnp.int32),
            pltpu.VMEM((_CCH, _D), jnp.float32),
            pltpu.VMEM((_CCH, _D), jnp.float32),
            pltpu.SemaphoreType.DMA,
            pltpu.SemaphoreType.DMA,
            pltpu.SemaphoreType.DMA,
            pltpu.SemaphoreType.DMA,
        ],
    )
    def k(y_hbm, dest_hbm, g0_hbm, g1_hbm, i0_v, i1_v, b0, b1, s0, s1, q0, q1):
        wid = lax.axis_index("s") * 2 + lax.axis_index("c")
        wb = [None, None]
        for c in range(_CROWS // _CCH):
            t0 = wid * _CROWS + c * _CCH
            pltpu.sync_copy(dest_hbm.at[0, pl.ds(t0, _CCH)], i0_v)
            pltpu.sync_copy(dest_hbm.at[1, pl.ds(t0, _CCH)], i1_v)
            cp0 = pltpu.async_copy(y_hbm.at[i0_v], b0, s0)
            cp1 = pltpu.async_copy(y_hbm.at[i1_v], b1, s1)
            cp0.wait()
            cp1.wait()
            if wb[0] is not None:
                wb[0].wait()
                wb[1].wait()
            wb[0] = pltpu.async_copy(b0, g0_hbm.at[pl.ds(t0, _CCH)], q0)
            wb[1] = pltpu.async_copy(b1, g1_hbm.at[pl.ds(t0, _CCH)], q1)
        wb[0].wait()
        wb[1].wait()

    return k(ys, dest2)


def _mix_body(g0_ref, g1_ref, w0_ref, w1_ref, out_ref):
    out_ref[...] = g0_ref[...] * w0_ref[...] + g1_ref[...] * w1_ref[...]


def _mix(g0, g1, wflat):
    grid_spec = pl.GridSpec(
        grid=(4,),
        in_specs=[
            pl.BlockSpec((_T // 4, _D), lambda i: (i, 0)),
            pl.BlockSpec((_T // 4, _D), lambda i: (i, 0)),
            pl.BlockSpec((_T // 4, 1), lambda i: (i, 0)),
            pl.BlockSpec((_T // 4, 1), lambda i: (4 + i, 0)),
        ],
        out_specs=pl.BlockSpec((_T // 4, _D), lambda i: (i, 0)),
    )
    return pl.pallas_call(
        _mix_body,
        grid_spec=grid_spec,
        out_shape=jax.ShapeDtypeStruct((_T, _D), jnp.float32),
    )(g0, g1, wflat, wflat)


def kernel(x, Wr, br, W1, b1, W2, b2):
    x2 = x.reshape(_T, _D)
    dest, wflat, te, tvr = _route(x2, Wr, br.reshape(1, _E))
    dest2 = dest.reshape(_K, _T)
    xs = _sc_dispatch(x2, dest2)
    ys = _ffn(xs, W1, b1, W2, b2, te.reshape(_NT), tvr.reshape(_NT))
    g0, g1 = _sc_combine(ys, dest2)
    out = _mix(g0, g1, wflat)
    return out.reshape(1, _T, _D)


# speedup vs baseline: 1.0248x; 1.0248x over previous
import functools

import jax
import jax.numpy as jnp
from jax import lax
from jax.experimental import pallas as pl
from jax.experimental.pallas import tpu as pltpu
from jax.experimental.pallas import tpu_sc as plsc

_T = 2048
_D = 1024
_H = 4096
_E = 8
_K = 2
_N = _T * _K
_TILE = 1024
_SUB = 256
_NSUB = _TILE // _SUB
_NT = _N // _TILE + _E
_NP = _NT * _TILE
_HC = 1024
_NH = _H // _HC
_NW = 32


def _route_body(x_ref, wr_ref, br_ref, dest_ref, w_ref, te_ref, ti_ref, tvr_ref):
    logits = lax.dot_general(x_ref[...], wr_ref[...], (((1,), (1,)), ((), ())),
                             preferred_element_type=jnp.float32) + br_ref[...]
    e_iota = lax.broadcasted_iota(jnp.int32, (_T, _E), 1)
    m0 = jnp.max(logits, axis=1, keepdims=True)
    i0 = jnp.min(jnp.where(logits == m0, e_iota, _E), axis=1, keepdims=True)
    masked = jnp.where(e_iota == i0, -jnp.inf, logits)
    m1 = jnp.max(masked, axis=1, keepdims=True)
    i1 = jnp.min(jnp.where(masked == m1, e_iota, _E), axis=1, keepdims=True)
    w0 = 1.0 / (1.0 + jnp.exp(m1 - m0))

    fid = jnp.concatenate([i0, i1], axis=0)
    lane_e = lax.broadcasted_iota(jnp.int32, (1, _E), 1)
    oh = (fid == lane_e).astype(jnp.int32)
    cs = oh
    sh = 1
    while sh < _N:
        cs = cs + jnp.concatenate(
            [jnp.zeros((sh, _E), jnp.int32), cs[:-sh, :]], axis=0)
        sh *= 2
    counts = cs[_N - 1:_N, :]
    rank = jnp.sum(oh * cs, axis=1, keepdims=True) - 1
    ptiles = lax.shift_right_logical(counts + (_TILE - 1), 10)
    pcs = ptiles
    for sh in (1, 2, 4):
        pcs = pcs + jnp.concatenate(
            [jnp.zeros((1, sh), jnp.int32), pcs[:, :-sh]], axis=1)
    gstart = (pcs - ptiles) * _TILE
    dest_ref[...] = jnp.sum(oh * gstart, axis=1, keepdims=True) + rank
    w_ref[...] = jnp.concatenate([w0, 1.0 - w0], axis=0)

    used = pcs[:, _E - 1:_E]
    ti = jnp.minimum(lax.broadcasted_iota(jnp.int32, (_NT, 1), 0),
                     used - 1)
    tstart = ti * _TILE
    te = jnp.clip(jnp.sum((gstart <= tstart).astype(jnp.int32),
                          axis=1, keepdims=True) - 1, 0, _E - 1)
    ohe = (te == lane_e).astype(jnp.int32)
    gct = jnp.sum(ohe * (gstart + counts), axis=1, keepdims=True)
    te_ref[...] = te
    ti_ref[...] = ti
    tvr_ref[...] = jnp.where(
        lax.broadcasted_iota(jnp.int32, (_NT, 1), 0) == ti,
        jnp.clip(gct - tstart, 0, _TILE), 0)


def _route(x2, Wr, br2):
    return pl.pallas_call(
        _route_body,
        out_shape=[
            jax.ShapeDtypeStruct((_N, 1), jnp.int32),
            jax.ShapeDtypeStruct((_N, 1), jnp.float32),
            jax.ShapeDtypeStruct((_NT, 1), jnp.int32),
            jax.ShapeDtypeStruct((_NT, 1), jnp.int32),
            jax.ShapeDtypeStruct((_NT, 1), jnp.int32),
        ],
    )(x2, Wr, br2)


_SLOTS = _N // _NW
_SCH = 32


def _sc_dispatch(x2, dest2):
    mesh = plsc.VectorSubcoreMesh(core_axis_name="c", subcore_axis_name="s")

    @functools.partial(
        pl.kernel,
        mesh=mesh,
        out_type=jax.ShapeDtypeStruct((_NP, _D), jnp.float32),
        scratch_types=[
            pltpu.VMEM((_SCH,), jnp.int32),
            pltpu.VMEM((_SCH,), jnp.int32),
            pltpu.VMEM((_SCH, _D), jnp.float32),
            pltpu.VMEM((_SCH, _D), jnp.float32),
            pltpu.SemaphoreType.DMA,
            pltpu.SemaphoreType.DMA,
        ],
    )
    def k(x_hbm, dest_hbm, xs_hbm, i0_v, i1_v, b0, b1, s0, s1):
        wid = lax.axis_index("s") * 2 + lax.axis_index("c")
        kk = wid // 16
        t0 = (wid % 16) * _SLOTS
        idxs = (i0_v, i1_v)
        bufs = (b0, b1)
        sems = (s0, s1)
        cps = [None, None]
        for c in range(_SLOTS // _SCH):
            b = c % 2
            if cps[b] is not None:
                cps[b].wait()
            pltpu.sync_copy(dest_hbm.at[kk, pl.ds(t0 + c * _SCH, _SCH)],
                            idxs[b])
            pltpu.sync_copy(x_hbm.at[pl.ds(t0 + c * _SCH, _SCH)], bufs[b])
            cps[b] = pltpu.async_copy(bufs[b], xs_hbm.at[idxs[b]], sems[b])
        for cp in cps:
            if cp is not None:
                cp.wait()

    return k(x2, dest2)


def _ffn_body(te_ref, ti_ref, tvr_ref, xs_ref, w1a_ref, w1b_ref, b1a_ref, b1b_ref,
              w2_ref, b2_ref, out_ref):
    i = pl.program_id(0)
    j = pl.program_id(1)
    w1a = w1a_ref[0].astype(jnp.bfloat16)
    w1b = w1b_ref[0].astype(jnp.bfloat16)
    w2 = w2_ref[0].astype(jnp.bfloat16)

    for k in range(_NSUB):
        @pl.when(tvr_ref[i] > k * _SUB)
        def _():
            rows = pl.ds(k * _SUB, _SUB)
            xb = xs_ref[rows, :].astype(jnp.bfloat16)
            h1 = lax.dot_general(xb, w1a, (((1,), (1,)), ((), ())),
                                 preferred_element_type=jnp.float32) + b1a_ref[0]
            h2 = lax.dot_general(xb, w1b, (((1,), (1,)), ((), ())),
                                 preferred_element_type=jnp.float32) + b1b_ref[0]
            a = (h1 * lax.logistic(h1) * h2).astype(jnp.bfloat16)
            part = lax.dot_general(a, w2, (((1,), (1,)), ((), ())),
                                   preferred_element_type=jnp.float32)

            @pl.when(j == 0)
            def _():
                out_ref[rows, :] = part

            @pl.when(j > 0)
            def _():
                out_ref[rows, :] = out_ref[rows, :] + part

            @pl.when(j == _NH - 1)
            def _():
                out_ref[rows, :] = out_ref[rows, :] + b2_ref[0]


def _ffn(xs, W1, b1, W2, b2, te, ti, tvr):
    grid_spec = pltpu.PrefetchScalarGridSpec(
        num_scalar_prefetch=3,
        grid=(_NT, _NH),
        in_specs=[
            pl.BlockSpec((_TILE, _D), lambda i, j, te, ti, tv: (ti[i], 0)),
            pl.BlockSpec((1, _HC, _D), lambda i, j, te, ti, tv: (te[i], j, 0)),
            pl.BlockSpec((1, _HC, _D),
                         lambda i, j, te, ti, tv: (te[i], _NH + j, 0)),
            pl.BlockSpec((1, 1, _HC), lambda i, j, te, ti, tv: (te[i], 0, j)),
            pl.BlockSpec((1, 1, _HC),
                         lambda i, j, te, ti, tv: (te[i], 0, _NH + j)),
            pl.BlockSpec((1, _D, _HC), lambda i, j, te, ti, tv: (te[i], 0, j)),
            pl.BlockSpec((1, 1, _D), lambda i, j, te, ti, tv: (te[i], 0, 0)),
        ],
        out_specs=pl.BlockSpec((_TILE, _D), lambda i, j, te, ti, tv: (ti[i], 0)),
    )
    return pl.pallas_call(
        _ffn_body,
        grid_spec=grid_spec,
        out_shape=jax.ShapeDtypeStruct((_NP, _D), jnp.float32),
    )(te, ti, tvr, xs, W1, W1, b1.reshape(_E, 1, 2 * _H), b1.reshape(_E, 1, 2 * _H),
      W2, b2.reshape(_E, 1, _D))


_CROWS = _T // _NW
_CCH = 32


def _sc_combine(ys, dest2):
    mesh = plsc.VectorSubcoreMesh(core_axis_name="c", subcore_axis_name="s")

    @functools.partial(
        pl.kernel,
        mesh=mesh,
        out_type=[
            jax.ShapeDtypeStruct((_T, _D), jnp.float32),
            jax.ShapeDtypeStruct((_T, _D), jnp.float32),
        ],
        scratch_types=[
            pltpu.VMEM((_CCH,), jnp.int32),
            pltpu.VMEM((_CCH,), jnp.int32),
            pltpu.VMEM((_CCH, _D), jnp.float32),
            pltpu.VMEM((_CCH, _D), jnp.float32),
            pltpu.SemaphoreType.DMA,
            pltpu.SemaphoreType.DMA,
            pltpu.SemaphoreType.DMA,
            pltpu.SemaphoreType.DMA,
        ],
    )
    def k(y_hbm, dest_hbm, g0_hbm, g1_hbm, i0_v, i1_v, b0, b1, s0, s1, q0, q1):
        wid = lax.axis_index("s") * 2 + lax.axis_index("c")
        wb = [None, None]
        for c in range(_CROWS // _CCH):
            t0 = wid * _CROWS + c * _CCH
            pltpu.sync_copy(dest_hbm.at[0, pl.ds(t0, _CCH)], i0_v)
            pltpu.sync_copy(dest_hbm.at[1, pl.ds(t0, _CCH)], i1_v)
            cp0 = pltpu.async_copy(y_hbm.at[i0_v], b0, s0)
            cp1 = pltpu.async_copy(y_hbm.at[i1_v], b1, s1)
            cp0.wait()
            cp1.wait()
            if wb[0] is not None:
                wb[0].wait()
                wb[1].wait()
            wb[0] = pltpu.async_copy(b0, g0_hbm.at[pl.ds(t0, _CCH)], q0)
            wb[1] = pltpu.async_copy(b1, g1_hbm.at[pl.ds(t0, _CCH)], q1)
        wb[0].wait()
        wb[1].wait()

    return k(ys, dest2)


def _mix_body(g0_ref, g1_ref, w0_ref, w1_ref, out_ref):
    out_ref[...] = g0_ref[...] * w0_ref[...] + g1_ref[...] * w1_ref[...]


def _mix(g0, g1, wflat):
    grid_spec = pl.GridSpec(
        grid=(4,),
        in_specs=[
            pl.BlockSpec((_T // 4, _D), lambda i: (i, 0)),
            pl.BlockSpec((_T // 4, _D), lambda i: (i, 0)),
            pl.BlockSpec((_T // 4, 1), lambda i: (i, 0)),
            pl.BlockSpec((_T // 4, 1), lambda i: (4 + i, 0)),
        ],
        out_specs=pl.BlockSpec((_T // 4, _D), lambda i: (i, 0)),
    )
    return pl.pallas_call(
        _mix_body,
        grid_spec=grid_spec,
        out_shape=jax.ShapeDtypeStruct((_T, _D), jnp.float32),
    )(g0, g1, wflat, wflat)


def kernel(x, Wr, br, W1, b1, W2, b2):
    x2 = x.reshape(_T, _D)
    dest, wflat, te, ti, tvr = _route(x2, Wr, br.reshape(1, _E))
    dest2 = dest.reshape(_K, _T)
    xs = _sc_dispatch(x2, dest2)
    ys = _ffn(xs, W1, b1, W2, b2, te.reshape(_NT), ti.reshape(_NT),
              tvr.reshape(_NT))
    g0, g1 = _sc_combine(ys, dest2)
    out = _mix(g0, g1, wflat)
    return out.reshape(1, _T, _D)
